# baseline (speedup 1.0000x reference)
import functools

import jax
import jax.numpy as jnp
from jax import lax
from jax.experimental import pallas as pl
from jax.experimental.pallas import tpu as pltpu
from jax.experimental.pallas import tpu_sc as plsc

B = 16384
D = 32
W = 128
NC = 2
NS = 16
NW = NC * NS
BPW = B // NW
L = 16
CHUNK = 256
NCHUNK = BPW // CHUNK


def _bpr_body(user_hbm, i_hbm, j_hbm, ut_hbm, it_hbm, out_hbm,
              idx_u, idx_i, idx_j, g_u, g_i, g_j,
              u_rows, i_rows, j_rows, out_v, sem):
    wid = lax.axis_index("s") * NC + lax.axis_index("c")
    base = wid * BPW

    pltpu.sync_copy(user_hbm.at[pl.ds(base, BPW)], idx_u)
    pltpu.sync_copy(i_hbm.at[pl.ds(base, BPW)], idx_i)
    pltpu.sync_copy(j_hbm.at[pl.ds(base, BPW)], idx_j)

    def shift_blk(t, _):
        sl = pl.ds(t * L, L)
        g_u[sl] = lax.shift_right_logical(idx_u[sl], 2)
        g_i[sl] = lax.shift_right_logical(idx_i[sl], 2)
        g_j[sl] = lax.shift_right_logical(idx_j[sl], 2)
        return 0

    lax.fori_loop(0, BPW // L, shift_blk, 0)

    for c in range(NCHUNK):
        csl = pl.ds(c * CHUNK, CHUNK)
        cu = pltpu.async_copy(ut_hbm.at[g_u.at[csl]], u_rows, sem)
        ci = pltpu.async_copy(it_hbm.at[g_i.at[csl]], i_rows, sem)
        cj = pltpu.async_copy(it_hbm.at[g_j.at[csl]], j_rows, sem)
        cu.wait()
        ci.wait()
        cj.wait()

        def block(blk, _):
            rows = blk * L + lax.iota(jnp.int32, L)
            gsl = pl.ds(c * CHUNK + blk * L, L)
            qu = jnp.left_shift(jnp.bitwise_and(idx_u[gsl], 3), 5)
            qi = jnp.left_shift(jnp.bitwise_and(idx_i[gsl], 3), 5)
            qj = jnp.left_shift(jnp.bitwise_and(idx_j[gsl], 3), 5)
            accs = [jnp.zeros((L,), jnp.float32) for _ in range(4)]
            for dd in range(D):
                u_v = plsc.load_gather(u_rows, [rows, qu + dd])
                i_v = plsc.load_gather(i_rows, [rows, qi + dd])
                j_v = plsc.load_gather(j_rows, [rows, qj + dd])
                accs[dd % 4] = accs[dd % 4] + u_v * (i_v - j_v)
            acc = (accs[0] + accs[1]) + (accs[2] + accs[3])
            out_v[pl.ds(c * CHUNK + blk * L, L)] = acc
            return 0

        lax.fori_loop(0, CHUNK // L, block, 0)

    pltpu.sync_copy(out_v, out_hbm.at[pl.ds(base, BPW)])


_bpr_kernel = functools.partial(
    pl.kernel,
    out_type=jax.ShapeDtypeStruct((B,), jnp.float32),
    mesh=plsc.VectorSubcoreMesh(core_axis_name="c", subcore_axis_name="s"),
    scratch_types=[
        pltpu.VMEM((BPW,), jnp.int32),
        pltpu.VMEM((BPW,), jnp.int32),
        pltpu.VMEM((BPW,), jnp.int32),
        pltpu.VMEM((BPW,), jnp.int32),
        pltpu.VMEM((BPW,), jnp.int32),
        pltpu.VMEM((BPW,), jnp.int32),
        pltpu.VMEM((CHUNK, W), jnp.float32),
        pltpu.VMEM((CHUNK, W), jnp.float32),
        pltpu.VMEM((CHUNK, W), jnp.float32),
        pltpu.VMEM((BPW,), jnp.float32),
        pltpu.SemaphoreType.DMA,
    ],
    compiler_params=pltpu.CompilerParams(
        needs_layout_passes=False, use_tc_tiling_on_sc=False),
)(_bpr_body)


def kernel(user, i, j, user_table, item_table):
    ut = user_table.reshape(user_table.shape[0] // 4, W)
    it = item_table.reshape(item_table.shape[0] // 4, W)
    return _bpr_kernel(user, i, j, ut, it)
